# Initial kernel scaffold; baseline (speedup 1.0000x reference)
import functools

import jax
import jax.numpy as jnp
from jax import lax
from jax.experimental import pallas as pl
from jax.experimental.pallas import tpu as pltpu
from jax.experimental.pallas import tpu_sc as plsc

_NC = 2
_NS = 16
_NW = _NC * _NS


@functools.partial(jax.jit, static_argnames=("chunk",))
def _gather_rows(idx, table, chunk=1024):
    (n,) = idx.shape
    _, d = table.shape
    b_per_w = n // _NW
    assert n % _NW == 0 and b_per_w % chunk == 0
    n_chunks = b_per_w // chunk

    mesh = plsc.VectorSubcoreMesh(core_axis_name="c", subcore_axis_name="s")

    @functools.partial(
        pl.kernel,
        out_type=jax.ShapeDtypeStruct((n, d), jnp.float32),
        mesh=mesh,
        scratch_types=[
            pltpu.VMEM((b_per_w,), jnp.int32),
            pltpu.VMEM((chunk, d), jnp.float32),
            pltpu.SemaphoreType.DMA,
        ],
    )
    def k(idx_hbm, table_hbm, out_hbm, idx_v, rows_v, sem):
        wid = lax.axis_index("s") * _NC + lax.axis_index("c")
        base = wid * b_per_w
        pltpu.sync_copy(idx_hbm.at[pl.ds(base, b_per_w)], idx_v)

        def body(j, carry):
            off = pl.multiple_of(j * chunk, chunk)
            pltpu.async_copy(
                table_hbm.at[idx_v.at[pl.ds(off, chunk)]], rows_v, sem
            ).wait()
            pltpu.sync_copy(rows_v, out_hbm.at[pl.ds(base + off, chunk)])
            return carry

        lax.fori_loop(0, n_chunks, body, 0)

    return k(idx, table)


def kernel(x, weight):
    b, h = x.shape
    _, d = weight.shape
    idx = x.reshape(b * h).astype(jnp.int32)
    out = _gather_rows(idx, weight)
    return out.reshape(b, h, d)


# speedup vs baseline: 1.1036x; 1.1036x over previous
import functools

import jax
import jax.numpy as jnp
from jax import lax
from jax.experimental import pallas as pl
from jax.experimental.pallas import tpu as pltpu
from jax.experimental.pallas import tpu_sc as plsc

_NC = 2
_NS = 16
_NW = _NC * _NS


@functools.partial(jax.jit, static_argnames=("chunk",))
def _gather_rows(idx, table, chunk=1024):
    (n,) = idx.shape
    _, d = table.shape
    b_per_w = n // _NW
    assert n % _NW == 0 and b_per_w % chunk == 0
    n_chunks = b_per_w // chunk

    mesh = plsc.VectorSubcoreMesh(core_axis_name="c", subcore_axis_name="s")

    @functools.partial(
        pl.kernel,
        out_type=jax.ShapeDtypeStruct((n, d), jnp.float32),
        mesh=mesh,
        compiler_params=pltpu.CompilerParams(use_tc_tiling_on_sc=False),
        scratch_types=[
            pltpu.VMEM((b_per_w,), jnp.int32),
            pltpu.VMEM((chunk, d), jnp.float32),
            pltpu.SemaphoreType.DMA,
        ],
    )
    def k(idx_hbm, table_hbm, out_hbm, idx_v, rows_v, sem):
        wid = lax.axis_index("s") * _NC + lax.axis_index("c")
        base = wid * b_per_w
        pltpu.sync_copy(idx_hbm.at[pl.ds(base, b_per_w)], idx_v)

        def body(j, carry):
            off = pl.multiple_of(j * chunk, chunk)
            pltpu.async_copy(
                table_hbm.at[idx_v.at[pl.ds(off, chunk)]], rows_v, sem
            ).wait()
            pltpu.sync_copy(rows_v, out_hbm.at[pl.ds(base + off, chunk)])
            return carry

        lax.fori_loop(0, n_chunks, body, 0)

    return k(idx, table)


def kernel(x, weight):
    b, h = x.shape
    _, d = weight.shape
    idx = x.reshape(b * h).astype(jnp.int32)
    out = _gather_rows(idx, weight)
    return out.reshape(b, h, d)


# speedup vs baseline: 1.1138x; 1.0092x over previous
import functools

import jax
import jax.numpy as jnp
from jax import lax
from jax.experimental import pallas as pl
from jax.experimental.pallas import tpu as pltpu
from jax.experimental.pallas import tpu_sc as plsc

_NC = 2
_NS = 16
_NW = _NC * _NS


@functools.partial(jax.jit, static_argnames=("chunk", "nbuf"))
def _gather_rows(idx, table, chunk=640, nbuf=4):
    (n,) = idx.shape
    _, d = table.shape
    b_per_w = n // _NW
    assert n % _NW == 0 and b_per_w % (chunk * nbuf) == 0
    n_chunks = b_per_w // chunk
    n_groups = n_chunks // nbuf

    mesh = plsc.VectorSubcoreMesh(core_axis_name="c", subcore_axis_name="s")

    @functools.partial(
        pl.kernel,
        out_type=jax.ShapeDtypeStruct((n, d), jnp.float32),
        mesh=mesh,
        compiler_params=pltpu.CompilerParams(use_tc_tiling_on_sc=False),
        scratch_types=[
            pltpu.VMEM((b_per_w,), jnp.int32),
            [pltpu.VMEM((chunk, d), jnp.float32) for _ in range(nbuf)],
            [pltpu.SemaphoreType.DMA for _ in range(nbuf)],
            [pltpu.SemaphoreType.DMA for _ in range(nbuf)],
        ],
    )
    def k(idx_hbm, table_hbm, out_hbm, idx_v, rows, gsem, ssem):
        wid = lax.axis_index("s") * _NC + lax.axis_index("c")
        base = wid * b_per_w
        pltpu.sync_copy(idx_hbm.at[pl.ds(base, b_per_w)], idx_v)

        def gather(j, b):
            off = pl.multiple_of(j * chunk, chunk)
            return pltpu.make_async_copy(
                table_hbm.at[idx_v.at[pl.ds(off, chunk)]], rows[b], gsem[b]
            )

        def store(j, b):
            off = pl.multiple_of(j * chunk, chunk)
            return pltpu.make_async_copy(
                rows[b], out_hbm.at[pl.ds(base + off, chunk)], ssem[b]
            )

        for b in range(nbuf):
            gather(b, b).start()

        def body(g, carry):
            for b in range(nbuf):
                j = g * nbuf + b
                gather(j, b).wait()
                off = pl.multiple_of(j * chunk, chunk)
                pltpu.sync_copy(rows[b], out_hbm.at[pl.ds(base + off, chunk)])
                jn = j + nbuf

                @pl.when(jn < n_chunks)
                def _():
                    gather(jn, b).start()

            return carry

        lax.fori_loop(0, n_groups, body, 0)

    return k(idx, table)


def kernel(x, weight):
    b, h = x.shape
    _, d = weight.shape
    idx = x.reshape(b * h).astype(jnp.int32)
    out = _gather_rows(idx, weight)
    return out.reshape(b, h, d)


# speedup vs baseline: 1.3545x; 1.2161x over previous
import functools

import jax
import jax.numpy as jnp
from jax import lax
from jax.experimental import pallas as pl
from jax.experimental.pallas import tpu as pltpu
from jax.experimental.pallas import tpu_sc as plsc

_NC = 2
_NS = 16
_NW = _NC * _NS
_L = 16


@jax.jit
def _gather_embed(idx, table):
    (n,) = idx.shape
    v, d = table.shape
    assert d == 32
    hist = 50
    b = n // hist
    assert b % (128 * _NW // 2) == 0 and hist % 2 == 0
    n_vb = b // 128
    vb_per_w = n_vb // (_NW // 2)
    h_per_w = hist // 2
    blk = vb_per_w * 128 * hist

    mesh = plsc.VectorSubcoreMesh(core_axis_name="c", subcore_axis_name="s")

    @functools.partial(
        pl.kernel,
        out_type=jax.ShapeDtypeStruct((hist, d // 8, n_vb, 8, 128),
                                      jnp.float32),
        mesh=mesh,
        compiler_params=pltpu.CompilerParams(
            use_tc_tiling_on_sc=False, needs_layout_passes=False),
        scratch_types=[
            pltpu.VMEM((blk,), jnp.int32),
            pltpu.VMEM((128,), jnp.int32),
            pltpu.VMEM((128, 32), jnp.float32),
            pltpu.VMEM((32, 128), jnp.float32),
            pltpu.SemaphoreType.DMA,
        ],
    )
    def k(idx_hbm, w_hbm, out_hbm, idxblk, lbuf, rbuf, obuf, sem):
        wid = lax.axis_index("s") * _NC + lax.axis_index("c")
        vbg = wid // 2
        hhalf = wid % 2
        pltpu.sync_copy(idx_hbm.at[pl.ds(vbg * blk, blk)], idxblk)

        lanes = lax.iota(jnp.int32, _L)
        iota_h = lanes * hist

        def unit(u, carry):
            h = hhalf * h_per_w + u // vb_per_w
            vb_l = u % vb_per_w
            base = vb_l * (128 * hist) + h
            for kb in range(8):
                addrs = iota_h + (base + kb * _L * hist)
                lbuf[pl.ds(kb * _L, _L)] = plsc.load_gather(idxblk, [addrs])
            pltpu.async_copy(w_hbm.at[lbuf], rbuf, sem).wait()

            def tr(dd, c2):
                col = jnp.full((_L,), dd, jnp.int32)
                for cb in range(8):
                    rows = lanes + cb * _L
                    obuf[dd, pl.ds(cb * _L, _L)] = plsc.load_gather(
                        rbuf, [rows, col])
                return c2

            lax.fori_loop(0, 32, tr, 0)

            vb = vbg * vb_per_w + vb_l
            for g in range(4):
                pltpu.sync_copy(obuf.at[pl.ds(g * 8, 8)],
                                out_hbm.at[h, g, vb])
            return carry

        lax.fori_loop(0, h_per_w * vb_per_w, unit, 0)

    return k(idx, table)


def kernel(x, weight):
    b, h = x.shape
    _, d = weight.shape
    idx = x.reshape(b * h).astype(jnp.int32)
    out5 = _gather_embed(idx, weight)
    return out5.transpose(2, 4, 0, 1, 3).reshape(b, h, d)


# speedup vs baseline: 1.6404x; 1.2111x over previous
import functools

import jax
import jax.numpy as jnp
from jax import lax
from jax.experimental import pallas as pl
from jax.experimental.pallas import tpu as pltpu
from jax.experimental.pallas import tpu_sc as plsc

_NC = 2
_NS = 16
_NW = _NC * _NS
_L = 16


@jax.jit
def _gather_embed(idx, table):
    (n,) = idx.shape
    v, d = table.shape
    assert d == 32
    hist = 50
    b = n // hist
    assert b % (128 * _NW // 2) == 0 and hist % 2 == 0
    n_vb = b // 128
    vb_per_w = n_vb // (_NW // 2)
    h_per_w = hist // 2
    blk = vb_per_w * 128 * hist
    n_units = h_per_w * 2

    mesh = plsc.VectorSubcoreMesh(core_axis_name="c", subcore_axis_name="s")

    @functools.partial(
        pl.kernel,
        out_type=jax.ShapeDtypeStruct((hist, d // 8, n_vb, 8, 128),
                                      jnp.float32),
        mesh=mesh,
        compiler_params=pltpu.CompilerParams(
            use_tc_tiling_on_sc=False, needs_layout_passes=False),
        scratch_types=[
            pltpu.VMEM((blk,), jnp.int32),
            pltpu.VMEM((512,), jnp.int32),
            pltpu.VMEM((512,), jnp.int32),
            pltpu.VMEM((512, 32), jnp.float32),
            pltpu.VMEM((512, 32), jnp.float32),
            pltpu.VMEM((4, 4, 8, 128), jnp.float32),
            pltpu.VMEM((4, 4, 8, 128), jnp.float32),
            pltpu.SemaphoreType.DMA,
            pltpu.SemaphoreType.DMA,
            pltpu.SemaphoreType.DMA,
            pltpu.SemaphoreType.DMA,
        ],
    )
    def k(idx_hbm, w_hbm, out_hbm, idxblk, lb0, lb1, rb0, rb1, ob0, ob1,
          gs0, gs1, ss0, ss1):
        wid = lax.axis_index("s") * _NC + lax.axis_index("c")
        vbg = wid // 2
        hhalf = wid % 2
        pltpu.sync_copy(idx_hbm.at[pl.ds(vbg * blk, blk)], idxblk)

        lanes = lax.iota(jnp.int32, _L)
        iota_h = lanes * hist
        lbufs = (lb0, lb1)
        rbufs = (rb0, rb1)
        obufs = (ob0, ob1)
        gsems = (gs0, gs1)
        ssems = (ss0, ss1)

        def unit_hq(u):
            h = hhalf * h_per_w + u // 2
            q = u % 2
            return h, q

        def build(u, s):
            h, q = unit_hq(u)
            base = q * (4 * 128 * hist) + h
            lb = lbufs[s]

            def bbody(kb, carry):
                addrs = iota_h + (base + kb * (_L * hist))
                off = pl.multiple_of(kb * _L, _L)
                lb[pl.ds(off, _L)] = plsc.load_gather(idxblk, [addrs])
                return carry

            lax.fori_loop(0, 32, bbody, 0)

        def gcopy(s):
            return pltpu.make_async_copy(w_hbm.at[lbufs[s]], rbufs[s],
                                         gsems[s])

        def transpose(s):
            rb, ob = rbufs[s], obufs[s]

            def tbody(dd, carry):
                col = jnp.full((_L,), dd, jnp.int32)
                g = dd // 8
                r = dd % 8
                for vbl in range(4):
                    for cb in range(8):
                        rows = lanes + (vbl * 128 + cb * _L)
                        ob[g, vbl, r, pl.ds(cb * _L, _L)] = (
                            plsc.load_gather(rb, [rows, col]))
                return carry

            lax.fori_loop(0, 32, tbody, 0)

        def scopies(u, s):
            h, q = unit_hq(u)
            vb0 = vbg * vb_per_w + q * 4
            return [
                pltpu.make_async_copy(
                    obufs[s].at[g], out_hbm.at[h, g, pl.ds(vb0, 4)], ssems[s])
                for g in range(4)
            ]

        build(0, 0)
        gcopy(0).start()

        def body(i, carry):
            u0 = i * 2
            u1 = u0 + 1

            build(u1, 1)
            gcopy(1).start()

            gcopy(0).wait()

            @pl.when(i > 0)
            def _():
                for c in scopies(u0 - 2, 0):
                    c.wait()

            transpose(0)
            for c in scopies(u0, 0):
                c.start()

            @pl.when(u0 + 2 < n_units)
            def _():
                build(u0 + 2, 0)
                gcopy(0).start()

            gcopy(1).wait()

            @pl.when(i > 0)
            def _():
                for c in scopies(u1 - 2, 1):
                    c.wait()

            transpose(1)
            for c in scopies(u1, 1):
                c.start()

            return carry

        lax.fori_loop(0, n_units // 2, body, 0)

        for c in scopies(n_units - 2, 0):
            c.wait()
        for c in scopies(n_units - 1, 1):
            c.wait()

    return k(idx, table)


def kernel(x, weight):
    b, h = x.shape
    _, d = weight.shape
    idx = x.reshape(b * h).astype(jnp.int32)
    out5 = _gather_embed(idx, weight)
    return out5.transpose(2, 4, 0, 1, 3).reshape(b, h, d)


# speedup vs baseline: 2.2385x; 1.3646x over previous
import functools

import jax
import jax.numpy as jnp
from jax import lax
from jax.experimental import pallas as pl
from jax.experimental.pallas import tpu as pltpu
from jax.experimental.pallas import tpu_sc as plsc

_NC = 2
_NS = 16
_NW = _NC * _NS
_L = 16


@jax.jit
def _gather_embed(idx, table):
    (n,) = idx.shape
    v4, d = table.shape
    assert d == 32
    hist = 50
    b = n // hist
    assert b % (128 * _NW // 2) == 0 and hist % 2 == 0
    n_vb = b // 128
    vb_per_w = n_vb // (_NW // 2)
    h_per_w = hist // 2
    blk = vb_per_w * 128 * hist
    n_units = h_per_w * 2

    mesh = plsc.VectorSubcoreMesh(core_axis_name="c", subcore_axis_name="s")

    @functools.partial(
        pl.kernel,
        out_type=jax.ShapeDtypeStruct((hist, d // 8, n_vb, 8, 128),
                                      jnp.float32),
        mesh=mesh,
        compiler_params=pltpu.CompilerParams(
            use_tc_tiling_on_sc=False, needs_layout_passes=False),
        scratch_types=[
            pltpu.VMEM((blk,), jnp.int32),
            pltpu.VMEM((512,), jnp.int32),
            pltpu.VMEM((512,), jnp.int32),
            pltpu.VMEM((512, 32), jnp.float32),
            pltpu.VMEM((512, 32), jnp.float32),
            pltpu.VMEM((4, 4, 8, 128), jnp.float32),
            pltpu.VMEM((4, 4, 8, 128), jnp.float32),
            pltpu.SemaphoreType.DMA,
            pltpu.SemaphoreType.DMA,
            pltpu.SemaphoreType.DMA,
            pltpu.SemaphoreType.DMA,
        ],
    )
    def k(idx_hbm, w_hbm, out_hbm, idxblk, lb0, lb1, rb0, rb1, ob0, ob1,
          gs0, gs1, ss0, ss1):
        wid = lax.axis_index("s") * _NC + lax.axis_index("c")
        vbg = wid // 2
        hhalf = wid % 2
        pltpu.sync_copy(idx_hbm.at[pl.ds(vbg * blk, blk)], idxblk)

        lanes = lax.iota(jnp.int32, _L)
        iota_h = lanes * hist
        lbufs = (lb0, lb1)
        rbufs = (rb0, rb1)
        obufs = (ob0, ob1)
        gsems = (gs0, gs1)
        ssems = (ss0, ss1)

        def unit_hq(u):
            h = hhalf * h_per_w + u // 2
            q = u % 2
            return h, q

        def build(u, s):
            h, q = unit_hq(u)
            base = q * (4 * 128 * hist) + h
            lb = lbufs[s]

            @plsc.parallel_loop(0, 32, unroll=4)
            def bbody(kb):
                addrs = iota_h + (base + kb * (_L * hist))
                off = pl.multiple_of(kb * _L, _L)
                lb[pl.ds(off, _L)] = plsc.load_gather(idxblk, [addrs]) * 4

        def gcopy(s):
            return pltpu.make_async_copy(w_hbm.at[lbufs[s]], rbufs[s],
                                         gsems[s])

        def transpose(s):
            rb, ob = rbufs[s], obufs[s]

            @plsc.parallel_loop(0, 32, unroll=4)
            def tbody(dd):
                col = jnp.full((_L,), dd, jnp.int32)
                g = dd // 8
                r = dd % 8
                for vbl in range(4):
                    for cb in range(8):
                        rows = lanes + (vbl * 128 + cb * _L)
                        ob[g, vbl, r, pl.ds(cb * _L, _L)] = (
                            plsc.load_gather(rb, [rows, col]))

        def scopies(u, s):
            h, q = unit_hq(u)
            vb0 = vbg * vb_per_w + q * 4
            return [
                pltpu.make_async_copy(
                    obufs[s].at[g], out_hbm.at[h, g, pl.ds(vb0, 4)], ssems[s])
                for g in range(4)
            ]

        build(0, 0)
        gcopy(0).start()

        def body(i, carry):
            u0 = i * 2
            u1 = u0 + 1

            build(u1, 1)
            gcopy(1).start()

            gcopy(0).wait()

            @pl.when(i > 0)
            def _():
                for c in scopies(u0 - 2, 0):
                    c.wait()

            transpose(0)
            for c in scopies(u0, 0):
                c.start()

            @pl.when(u0 + 2 < n_units)
            def _():
                build(u0 + 2, 0)
                gcopy(0).start()

            gcopy(1).wait()

            @pl.when(i > 0)
            def _():
                for c in scopies(u1 - 2, 1):
                    c.wait()

            transpose(1)
            for c in scopies(u1, 1):
                c.start()

            return carry

        lax.fori_loop(0, n_units // 2, body, 0)

        for c in scopies(n_units - 2, 0):
            c.wait()
        for c in scopies(n_units - 1, 1):
            c.wait()

    return k(idx, table)


def kernel(x, weight):
    b, h = x.shape
    v, d = weight.shape
    idx = x.reshape(b * h).astype(jnp.int32)
    wp = jnp.pad(weight, ((0, 0), (0, 128 - d))).reshape(v * (128 // d), d)
    out5 = _gather_embed(idx, wp)
    return out5.transpose(2, 4, 0, 1, 3).reshape(b, h, d)
